# baseline (speedup 1.0000x reference)
import functools

import jax
import jax.numpy as jnp
from jax import lax
from jax.experimental import pallas as pl
from jax.experimental.pallas import tpu as pltpu
from jax.experimental.pallas import tpu_sc as plsc

B, T = 16384, 200
V = 260
D = 64
NC, NS = 2, 16
NW = NC * NS
NU = T * (D // 8)
UPW = NU // NW
NBC = B // 128
CHUNK = 32
NCHUNK = NBC // CHUNK
UNROLL = 8

_mesh = plsc.VectorSubcoreMesh(core_axis_name="c", subcore_axis_name="s")


@functools.partial(
    pl.kernel,
    out_type=jax.ShapeDtypeStruct((NU, NBC, 8, 128), jnp.float32),
    mesh=_mesh,
    compiler_params=pltpu.CompilerParams(
        use_tc_tiling_on_sc=False, needs_layout_passes=False
    ),
    scratch_types=[
        pltpu.VMEM((B,), jnp.int32),
        pltpu.VMEM((2, CHUNK, 8, 128), jnp.float32),
        pltpu.VMEM((V * D,), jnp.float32),
        pltpu.SemaphoreType.DMA,
        pltpu.SemaphoreType.DMA,
    ],
)
def _embed_sc(xt_hbm, tab_hbm, out_hbm, idx_v, stage_v, tab_v, sem0, sem1):
    wid = lax.axis_index("s") * NC + lax.axis_index("c")
    u0 = wid * UPW

    pltpu.sync_copy(tab_hbm, tab_v)
    sems = (sem0, sem1)

    def fill_chunk(buf, chunk_base, rowbase):
        @plsc.parallel_loop(0, CHUNK, 1, unroll=UNROLL)
        def bc_body(bi):
            base = chunk_base + bi * 128
            for b16 in range(8):
                idx16 = idx_v[pl.ds(base + b16 * 16, 16)]
                for d_ in range(8):
                    flat16 = idx16 + rowbase[d_]
                    vals = plsc.load_gather(tab_v, [flat16])
                    stage_v[buf, bi, d_, pl.ds(b16 * 16, 16)] = vals

    def wait_buf(buf):
        pltpu.make_async_copy(
            stage_v.at[buf],
            out_hbm.at[0, pl.ds(0, CHUNK)],
            sems[buf],
        ).wait()

    for buf in (0, 1):
        pltpu.async_copy(
            stage_v.at[buf],
            out_hbm.at[u0, pl.ds(buf * CHUNK, CHUNK)],
            sems[buf],
        )

    def unit_body(k, carry):
        u = u0 + k
        t = u // 8
        dr8 = (u % 8) * 8
        rowbase = [
            jnp.full((16,), (dr8 + d_) * V, jnp.int32) for d_ in range(8)
        ]

        @pl.when(jnp.logical_or(k == 0, u % 8 == 0))
        def _load_idx():
            pltpu.sync_copy(xt_hbm.at[t], idx_v)

        for chunk in range(NCHUNK):
            buf = chunk % 2
            wait_buf(buf)
            fill_chunk(buf, chunk * (CHUNK * 128), rowbase)
            pltpu.async_copy(
                stage_v.at[buf],
                out_hbm.at[u, pl.ds(chunk * CHUNK, CHUNK)],
                sems[buf],
            )
        return carry

    lax.fori_loop(0, UPW, unit_body, 0)
    wait_buf(0)
    wait_buf(1)


def kernel(x, embed_param):
    xt = x.T.reshape(T, B)
    tab = embed_param.T.reshape(D * V)
    out5 = _embed_sc(xt, tab)
    out = out5.reshape(T, 8, NBC, 8, 128)
    out = out.transpose(2, 4, 0, 1, 3)
    return out.reshape(B, T, D)


# speedup vs baseline: 1.2925x; 1.2925x over previous
import functools

import jax
import jax.numpy as jnp
from jax import lax
from jax.experimental import pallas as pl
from jax.experimental.pallas import tpu as pltpu
from jax.experimental.pallas import tpu_sc as plsc

B, T = 16384, 200
V = 260
D = 64
NC, NS = 2, 16
NW = NC * NS
NU = T * (D // 8)
UPW = NU // NW
NBC = B // 128
CHUNK = 32
NCHUNK = NBC // CHUNK
UNROLL = 4

_mesh = plsc.VectorSubcoreMesh(core_axis_name="c", subcore_axis_name="s")


@functools.partial(
    pl.kernel,
    out_type=jax.ShapeDtypeStruct((NU, NBC, 8, 128), jnp.float32),
    mesh=_mesh,
    compiler_params=pltpu.CompilerParams(
        use_tc_tiling_on_sc=False, needs_layout_passes=False
    ),
    scratch_types=[
        pltpu.VMEM((B,), jnp.int32),
        pltpu.VMEM((2, CHUNK, 8, 128), jnp.float32),
        pltpu.VMEM((V * D,), jnp.float32),
        pltpu.SemaphoreType.DMA,
        pltpu.SemaphoreType.DMA,
    ],
)
def _embed_sc(xt_hbm, tab_hbm, out_hbm, idx_v, stage_v, tab_v, sem0, sem1):
    wid = lax.axis_index("s") * NC + lax.axis_index("c")
    u0 = wid * UPW

    pltpu.sync_copy(tab_hbm, tab_v)
    sems = (sem0, sem1)

    def fill_chunk(buf, chunk_base, rowbase):
        @plsc.parallel_loop(0, CHUNK, 1, unroll=UNROLL)
        def bc_body(bi):
            base = chunk_base + bi * 128
            for b16 in range(8):
                idx16 = idx_v[pl.ds(base + b16 * 16, 16)]
                for d_ in range(8):
                    flat16 = idx16 + rowbase[d_]
                    vals = plsc.load_gather(tab_v, [flat16])
                    stage_v[buf, bi, d_, pl.ds(b16 * 16, 16)] = vals

    def wait_buf(buf):
        pltpu.make_async_copy(
            stage_v.at[buf],
            out_hbm.at[0, pl.ds(0, CHUNK)],
            sems[buf],
        ).wait()

    for buf in (0, 1):
        pltpu.async_copy(
            stage_v.at[buf],
            out_hbm.at[u0, pl.ds(buf * CHUNK, CHUNK)],
            sems[buf],
        )

    def unit_body(k, carry):
        u = u0 + k
        t = u // 8
        dr8 = (u % 8) * 8
        rowbase = [
            jnp.full((16,), (dr8 + d_) * V, jnp.int32) for d_ in range(8)
        ]

        @pl.when(jnp.logical_or(k == 0, u % 8 == 0))
        def _load_idx():
            pltpu.sync_copy(xt_hbm.at[t], idx_v)

        for chunk in range(NCHUNK):
            buf = chunk % 2
            wait_buf(buf)
            fill_chunk(buf, chunk * (CHUNK * 128), rowbase)
            pltpu.async_copy(
                stage_v.at[buf],
                out_hbm.at[u, pl.ds(chunk * CHUNK, CHUNK)],
                sems[buf],
            )
        return carry

    lax.fori_loop(0, UPW, unit_body, 0)
    wait_buf(0)
    wait_buf(1)


def kernel(x, embed_param):
    xt = x.T.reshape(T, B)
    tab = embed_param.T.reshape(D * V)
    out5 = _embed_sc(xt, tab)
    out = out5.reshape(T, 8, NBC, 8, 128)
    out = out.transpose(2, 4, 0, 1, 3)
    return out.reshape(B, T, D)


# speedup vs baseline: 1.4310x; 1.1071x over previous
import functools

import jax
import jax.numpy as jnp
from jax import lax
from jax.experimental import pallas as pl
from jax.experimental.pallas import tpu as pltpu
from jax.experimental.pallas import tpu_sc as plsc

B, T = 16384, 200
V = 260
D = 64
NC, NS = 2, 16
NW = NC * NS
NU = T * (D // 8)
UPW = NU // NW
NBC = B // 128
CHUNK = 32
NCHUNK = NBC // CHUNK
UNROLL = 4

_mesh = plsc.VectorSubcoreMesh(core_axis_name="c", subcore_axis_name="s")


@functools.partial(
    pl.kernel,
    out_type=jax.ShapeDtypeStruct((NU, NBC, 8, 128), jnp.float32),
    mesh=_mesh,
    compiler_params=pltpu.CompilerParams(
        use_tc_tiling_on_sc=False, needs_layout_passes=False
    ),
    scratch_types=[
        pltpu.VMEM((B,), jnp.int32),
        pltpu.VMEM((2, CHUNK, 8, 128), jnp.float32),
        pltpu.VMEM((D, V), jnp.float32),
        pltpu.SemaphoreType.DMA,
        pltpu.SemaphoreType.DMA,
    ],
)
def _embed_sc(xt_hbm, tab_hbm, out_hbm, idx_v, stage_v, tab_v, sem0, sem1):
    wid = lax.axis_index("s") * NC + lax.axis_index("c")
    u0 = wid * UPW

    pltpu.sync_copy(tab_hbm, tab_v)
    sems = (sem0, sem1)

    def fill_chunk(buf, chunk_base, rowrefs):
        @plsc.parallel_loop(0, CHUNK, 1, unroll=UNROLL)
        def bc_body(bi):
            base = chunk_base + bi * 128
            for b16 in range(8):
                idx16 = idx_v[pl.ds(base + b16 * 16, 16)]
                for d_ in range(8):
                    vals = plsc.load_gather(rowrefs[d_], [idx16])
                    stage_v[buf, bi, d_, pl.ds(b16 * 16, 16)] = vals

    def wait_buf(buf):
        pltpu.make_async_copy(
            stage_v.at[buf],
            out_hbm.at[0, pl.ds(0, CHUNK)],
            sems[buf],
        ).wait()

    for buf in (0, 1):
        pltpu.async_copy(
            stage_v.at[buf],
            out_hbm.at[u0, pl.ds(buf * CHUNK, CHUNK)],
            sems[buf],
        )

    def unit_body(k, carry):
        u = u0 + k
        t = u // 8
        dr8 = (u % 8) * 8
        rowrefs = [tab_v.at[dr8 + d_] for d_ in range(8)]

        @pl.when(jnp.logical_or(k == 0, u % 8 == 0))
        def _load_idx():
            pltpu.sync_copy(xt_hbm.at[t], idx_v)

        for chunk in range(NCHUNK):
            buf = chunk % 2
            wait_buf(buf)
            fill_chunk(buf, chunk * (CHUNK * 128), rowrefs)
            pltpu.async_copy(
                stage_v.at[buf],
                out_hbm.at[u, pl.ds(chunk * CHUNK, CHUNK)],
                sems[buf],
            )
        return carry

    lax.fori_loop(0, UPW, unit_body, 0)
    wait_buf(0)
    wait_buf(1)


def kernel(x, embed_param):
    xt = x.T.reshape(T, B)
    tab = embed_param.T.reshape(D, V)
    out5 = _embed_sc(xt, tab)
    out = out5.reshape(T, 8, NBC, 8, 128)
    out = out.transpose(2, 4, 0, 1, 3)
    return out.reshape(B, T, D)
